# baseline (speedup 1.0000x reference)
import jax
import jax.numpy as jnp
from jax import lax
from jax.experimental import pallas as pl
from jax.experimental.pallas import tpu as pltpu
from jax.experimental.pallas import tpu_sc as plsc

D = 64
B, T = 4096, 200
NC, NS = 2, 16
NW = NC * NS
BL = 128
NBUF = 4
SCALE = 8.0
VPB = BL // 16


def _body(table_hbm, idx_hbm, out_hbm, idx_bufs, pidx_bufs, hoff_bufs,
          pair_bufs, out_bufs, gsems, ssems):
    wid = lax.axis_index("s") * NC + lax.axis_index("c")
    b0 = wid * BL

    def fire_gathers(t, b):
        pltpu.sync_copy(idx_hbm.at[t, pl.ds(b0, BL)], idx_bufs[b])
        for k in range(VPB):
            v = idx_bufs[b][pl.ds(k * 16, 16)]
            pidx_bufs[b][pl.ds(k * 16, 16)] = lax.shift_right_logical(v, 1)
            hoff_bufs[b][pl.ds(k * 16, 16)] = lax.shift_left(
                lax.bitwise_and(v, 1), 6)
        pltpu.async_copy(
            table_hbm.at[pidx_bufs[b]], pair_bufs[b], gsems[b])

    def wait_gather(b):
        pltpu.make_async_copy(
            table_hbm.at[pidx_bufs[b]], pair_bufs[b], gsems[b]).wait()

    def wait_store(b):
        pltpu.make_async_copy(
            out_bufs[b], out_hbm.at[0, :, 0], ssems[b]).wait()

    for b in range(NBUF - 1):
        fire_gathers(b, b)

    def chunk_iter(s, carry):
        for b in range(NBUF):
            t = s * NBUF + b
            wait_gather(b)

            def tgroup(j, c):
                rows = jax.lax.iota(jnp.int32, 16) + j * 16
                cols0 = hoff_bufs[b][pl.ds(j * 16, 16)]
                for d in range(D):
                    v = plsc.load_gather(pair_bufs[b], [rows, cols0 + d])
                    out_bufs[b][d // 8, d % 8, pl.ds(j * 16, 16)] = v * SCALE
                return c

            lax.fori_loop(0, VPB, tgroup, 0)

            pltpu.async_copy(out_bufs[b], out_hbm.at[t, :, wid], ssems[b])

            bb = (b + NBUF - 1) % NBUF

            @pl.when(t + NBUF - 1 < T)
            def _prime():
                @pl.when(t >= 1)
                def _drain_store():
                    wait_store(bb)

                fire_gathers(t + NBUF - 1, bb)

        return carry

    lax.fori_loop(0, T // NBUF, chunk_iter, 0)

    for b in range(NBUF):
        wait_store(b)


@jax.jit
def _emb(table2, idxT):
    mesh = plsc.VectorSubcoreMesh(core_axis_name="c", subcore_axis_name="s")
    return pl.kernel(
        _body,
        out_type=jax.ShapeDtypeStruct((T, 8, NW, 8, BL), jnp.float32),
        mesh=mesh,
        compiler_params=pltpu.CompilerParams(needs_layout_passes=False),
        scratch_types=[
            [pltpu.VMEM((BL,), jnp.int32) for _ in range(NBUF)],
            [pltpu.VMEM((BL,), jnp.int32) for _ in range(NBUF)],
            [pltpu.VMEM((BL,), jnp.int32) for _ in range(NBUF)],
            [pltpu.VMEM((BL, 128), jnp.float32) for _ in range(NBUF)],
            [pltpu.VMEM((8, 8, BL), jnp.float32) for _ in range(NBUF)],
            [pltpu.SemaphoreType.DMA for _ in range(NBUF)],
            [pltpu.SemaphoreType.DMA for _ in range(NBUF)],
        ],
    )(table2, idxT)


def kernel(x, table):
    table2 = table.reshape(500000, 128)
    idxT = x.T
    out5d = _emb(table2, idxT)
    return out5d.transpose(2, 4, 0, 1, 3).reshape(B, T, D)
